# Initial kernel scaffold; baseline (speedup 1.0000x reference)
import jax
import jax.numpy as jnp
from jax.experimental import pallas as pl


def kernel(id, map_param):
    raise NotImplementedError("write your pallas kernel here")


# speedup vs baseline: 1.1065x; 1.1065x over previous
import functools

import jax
import jax.numpy as jnp
from jax import lax
from jax.experimental import pallas as pl
from jax.experimental.pallas import tpu as pltpu
from jax.experimental.pallas import tpu_sc as plsc

_BATCH = 16384
_NC = 2
_NS = 16
_NW = _NC * _NS
_CHUNK = 128
_ROWS_PER_W = _BATCH // (_NW * _CHUNK)

@functools.cache
def _build_gather_sc():
    mesh = plsc.VectorSubcoreMesh(core_axis_name="c", subcore_axis_name="s")

    @functools.partial(
        pl.kernel,
        mesh=mesh,
        out_type=jax.ShapeDtypeStruct((_NW * _ROWS_PER_W, _CHUNK), jnp.float32),
        scratch_types=[
            pltpu.VMEM((_ROWS_PER_W, _CHUNK), jnp.int32),
            pltpu.VMEM((_ROWS_PER_W, _CHUNK), jnp.float32),
            pltpu.SemaphoreType.DMA,
        ],
    )
    def _gather_sc(idx_hbm, table_hbm, out_hbm, idx_v, vals_v, sem):
        wid = lax.axis_index("s") * _NC + lax.axis_index("c")
        base = wid * _ROWS_PER_W
        pltpu.sync_copy(idx_hbm.at[pl.ds(base, _ROWS_PER_W)], idx_v)
        copies = [
            pltpu.async_copy(table_hbm.at[idx_v.at[j]], vals_v.at[j], sem)
            for j in range(_ROWS_PER_W)
        ]
        for c in copies:
            c.wait()
        pltpu.sync_copy(vals_v, out_hbm.at[pl.ds(base, _ROWS_PER_W)])

    return _gather_sc


def kernel(id, map_param):
    idx2d = id.astype(jnp.int32).reshape(_NW * _ROWS_PER_W, _CHUNK)
    out = _build_gather_sc()(idx2d, map_param)
    return out.reshape(_BATCH)


# speedup vs baseline: 1.1079x; 1.0013x over previous
import functools

import jax
import jax.numpy as jnp
from jax import lax
from jax.experimental import pallas as pl
from jax.experimental.pallas import tpu as pltpu
from jax.experimental.pallas import tpu_sc as plsc

_BATCH = 16384
_NC = 2
_NS = 16
_NW = _NC * _NS
_PER_W = _BATCH // _NW


@functools.cache
def _build_gather_sc():
    mesh = plsc.VectorSubcoreMesh(core_axis_name="c", subcore_axis_name="s")

    @functools.partial(
        pl.kernel,
        mesh=mesh,
        out_type=jax.ShapeDtypeStruct((_BATCH,), jnp.float32),
        scratch_types=[
            pltpu.VMEM((_PER_W,), jnp.int32),
            pltpu.VMEM((_PER_W,), jnp.float32),
            pltpu.SemaphoreType.DMA,
        ],
    )
    def _gather_sc(idx_hbm, table_hbm, out_hbm, idx_v, vals_v, sem):
        wid = lax.axis_index("s") * _NC + lax.axis_index("c")
        base = wid * _PER_W
        pltpu.sync_copy(idx_hbm.at[pl.ds(base, _PER_W)], idx_v)
        pltpu.async_copy(table_hbm.at[idx_v], vals_v, sem).wait()
        pltpu.sync_copy(vals_v, out_hbm.at[pl.ds(base, _PER_W)])

    return _gather_sc


def kernel(id, map_param):
    return _build_gather_sc()(id.astype(jnp.int32), map_param)


# speedup vs baseline: 1.1459x; 1.0343x over previous
import functools

import jax
import jax.numpy as jnp
from jax import lax
from jax.experimental import pallas as pl
from jax.experimental.pallas import tpu as pltpu
from jax.experimental.pallas import tpu_sc as plsc

_BATCH = 16384
_NC = 1
_NS = 16
_NW = _NC * _NS
_PER_W = _BATCH // _NW


@functools.cache
def _build_gather_sc():
    mesh = plsc.VectorSubcoreMesh(core_axis_name="c", subcore_axis_name="s", num_cores=1)

    @functools.partial(
        pl.kernel,
        mesh=mesh,
        out_type=jax.ShapeDtypeStruct((_BATCH,), jnp.float32),
        scratch_types=[
            pltpu.VMEM((_PER_W,), jnp.int32),
            pltpu.VMEM((_PER_W,), jnp.float32),
            pltpu.SemaphoreType.DMA,
        ],
    )
    def _gather_sc(idx_hbm, table_hbm, out_hbm, idx_v, vals_v, sem):
        wid = lax.axis_index("s") * _NC + lax.axis_index("c")
        base = wid * _PER_W
        pltpu.sync_copy(idx_hbm.at[pl.ds(base, _PER_W)], idx_v)
        pltpu.async_copy(table_hbm.at[idx_v], vals_v, sem).wait()
        pltpu.sync_copy(vals_v, out_hbm.at[pl.ds(base, _PER_W)])

    return _gather_sc


def kernel(id, map_param):
    return _build_gather_sc()(id.astype(jnp.int32), map_param)


# speedup vs baseline: 1.1476x; 1.0015x over previous
import functools

import jax
import jax.numpy as jnp
from jax import lax
from jax.experimental import pallas as pl
from jax.experimental.pallas import tpu as pltpu
from jax.experimental.pallas import tpu_sc as plsc

_BATCH = 16384
_NC = 1
_NS = 16
_NW = _NC * _NS
_PER_W = _BATCH // _NW


@functools.cache
def _build_gather_sc():
    mesh = plsc.VectorSubcoreMesh(core_axis_name="c", subcore_axis_name="s", num_cores=1)

    @functools.partial(
        pl.kernel,
        mesh=mesh,
        out_type=jax.ShapeDtypeStruct((_BATCH,), jnp.float32),
        scratch_types=[
            pltpu.VMEM((_PER_W,), jnp.int32),
            pltpu.VMEM((_PER_W,), jnp.float32),
            pltpu.SemaphoreType.DMA,
        ],
    )
    def _gather_sc(idx_hbm, table_hbm, out_hbm, idx_v, vals_v, sem):
        wid = lax.axis_index("s") * _NC + lax.axis_index("c")
        base = wid * _PER_W
        pltpu.sync_copy(idx_hbm.at[pl.ds(base, _PER_W)], idx_v)
        h = _PER_W // 2
        c0 = pltpu.async_copy(table_hbm.at[idx_v.at[pl.ds(0, h)]], vals_v.at[pl.ds(0, h)], sem)
        c1 = pltpu.async_copy(table_hbm.at[idx_v.at[pl.ds(h, h)]], vals_v.at[pl.ds(h, h)], sem)
        c0.wait()
        c1.wait()
        pltpu.sync_copy(vals_v, out_hbm.at[pl.ds(base, _PER_W)])

    return _gather_sc


def kernel(id, map_param):
    return _build_gather_sc()(id.astype(jnp.int32), map_param)
